# Initial kernel scaffold; baseline (speedup 1.0000x reference)
import jax
import jax.numpy as jnp
from jax.experimental import pallas as pl


def kernel(a, b):
    raise NotImplementedError("write your pallas kernel here")


# speedup vs baseline: 1.1887x; 1.1887x over previous
import functools

import numpy as np
import jax
import jax.numpy as jnp
from jax import lax
from jax.experimental import pallas as pl
from jax.experimental.pallas import tpu as pltpu
from jax.experimental.pallas import tpu_sc as plsc

N_COLS = 100000
N_S = 4096
N_CH = 3
NW = 32
SLOTS = N_CH * N_S
PER_W = SLOTS // NW
ROW_CHUNK = 128
N_RCHUNK = PER_W // ROW_CHUNK
N_GRP = PER_W // 16
N_BCHUNK = N_S // 8


def _build_tables():
    rng = np.random.default_rng(0)
    a_idx = rng.permutation(N_COLS)[:N_S].astype(np.int64)
    b_idx = rng.permutation(N_COLS)[:N_S].astype(np.int64)

    def tables(idx, slot_to_k):
        m = np.arange(SLOTS, dtype=np.int64)
        c = m // N_S
        k = slot_to_k(m % N_S)
        elem = c * N_COLS + idx[k]
        row = (elem // 16).astype(np.int32)
        col = (elem % 16).astype(np.int32)
        o = np.tile(np.arange(PER_W, dtype=np.int32), NW)
        sel = o * 16 + col
        return (row.reshape(NW, N_RCHUNK, ROW_CHUNK),
                sel.reshape(NW, N_GRP, 16))

    row_a, sel_a = tables(a_idx, lambda r: r)
    row_b, sel_b = tables(b_idx, lambda r: r)
    return row_a, sel_a, row_b, sel_b


_ROW_A, _SEL_A, _ROW_B, _SEL_B = _build_tables()


def _sc_gather_body(a_hbm, b_hbm, row_a, sel_a, row_b, sel_b,
                    out_a, out_b, idx_v, rows_v, sel_v, out_v, sem):
    wid = lax.axis_index("s") * 2 + lax.axis_index("c")

    def gather_side(src_hbm, row_hbm, sel_hbm, dst_hbm):
        pltpu.sync_copy(row_hbm.at[wid], idx_v)
        pltpu.sync_copy(sel_hbm.at[wid], sel_v)
        for j in range(N_RCHUNK):
            pltpu.async_copy(
                src_hbm.at[idx_v.at[j]],
                rows_v.at[pl.ds(j * ROW_CHUNK, ROW_CHUNK)],
                sem,
            ).wait()
        for g in range(N_GRP):
            s = sel_v[g]
            r = lax.shift_right_logical(s, 4)
            col = lax.bitwise_and(s, 15)
            out_v[pl.ds(g * 16, 16)] = plsc.load_gather(rows_v, [r, col])
        pltpu.sync_copy(out_v, dst_hbm.at[pl.ds(wid * PER_W, PER_W)])

    gather_side(a_hbm, row_a, sel_a, out_a)
    gather_side(b_hbm, row_b, sel_b, out_b)


@functools.cache
def _sc_gather():
    return pl.kernel(
        _sc_gather_body,
        mesh=plsc.VectorSubcoreMesh(core_axis_name="c", subcore_axis_name="s"),
        compiler_params=pltpu.CompilerParams(
            needs_layout_passes=False, use_tc_tiling_on_sc=False),
        out_type=[
            jax.ShapeDtypeStruct((SLOTS,), jnp.float32),
            jax.ShapeDtypeStruct((SLOTS,), jnp.float32),
        ],
        scratch_types=[
            pltpu.VMEM((N_RCHUNK, ROW_CHUNK), jnp.int32),
            pltpu.VMEM((PER_W, 16), jnp.float32),
            pltpu.VMEM((N_GRP, 16), jnp.int32),
            pltpu.VMEM((PER_W,), jnp.float32),
            pltpu.SemaphoreType.DMA,
        ],
    )


def _tc_chamfer_body(a_ref, b_ref, out_ref):
    a0 = a_ref[0:1, :]
    a1 = a_ref[1:2, :]
    a2 = a_ref[2:3, :]

    def step(t, acc):
        b0 = b_ref[0, pl.ds(t * 8, 8), :]
        b1 = b_ref[1, pl.ds(t * 8, 8), :]
        b2 = b_ref[2, pl.ds(t * 8, 8), :]
        d = (a0 - b0) ** 2 + (a1 - b1) ** 2 + (a2 - b2) ** 2
        return jnp.minimum(acc, d)

    acc = lax.fori_loop(
        0, N_BCHUNK, step,
        jnp.full((8, N_S), jnp.inf, dtype=jnp.float32),
    )
    out_ref[0, 0] = jnp.sum(jnp.min(acc, axis=0))


_tc_chamfer = pl.pallas_call(
    _tc_chamfer_body,
    out_shape=jax.ShapeDtypeStruct((1, 1), jnp.float32),
    in_specs=[
        pl.BlockSpec(memory_space=pltpu.VMEM),
        pl.BlockSpec(memory_space=pltpu.VMEM),
    ],
    out_specs=pl.BlockSpec(memory_space=pltpu.SMEM),
)


@jax.jit
def kernel(a, b):
    a_flat = a.reshape(N_COLS * 6 // 16, 16)
    b_flat = b.reshape(N_COLS * 6 // 16, 16)
    ag_flat, bg_flat = _sc_gather()(
        a_flat, b_flat,
        jnp.asarray(_ROW_A), jnp.asarray(_SEL_A),
        jnp.asarray(_ROW_B), jnp.asarray(_SEL_B),
    )
    a_g = ag_flat.reshape(N_CH, N_S)
    b_g = bg_flat.reshape(N_CH, N_S, 1)
    return _tc_chamfer(a_g, b_g)[0, 0]


# speedup vs baseline: 1.5860x; 1.3343x over previous
import functools

import numpy as np
import jax
import jax.numpy as jnp
from jax import lax
from jax.experimental import pallas as pl
from jax.experimental.pallas import tpu as pltpu
from jax.experimental.pallas import tpu_sc as plsc

N_COLS = 100000
N_S = 4096
N_CH = 3
NW = 32
S_PER_W = N_S // NW
PER_W = N_CH * S_PER_W
ROW_CHUNK = 128
N_RCHUNK = PER_W // ROW_CHUNK
N_GRP = PER_W // 16


def _build_tables():
    rng = np.random.default_rng(0)
    a_idx = rng.permutation(N_COLS)[:N_S].astype(np.int64)
    b_idx = rng.permutation(N_COLS)[:N_S].astype(np.int64)

    def tables(idx):
        m = np.arange(NW * PER_W, dtype=np.int64)
        w, o = m // PER_W, m % PER_W
        c, p = o // S_PER_W, o % S_PER_W
        elem = c * N_COLS + idx[w * S_PER_W + p]
        row = (elem // 16).astype(np.int32)
        sel = (o * 16 + elem % 16).astype(np.int32)
        return (row.reshape(NW, N_RCHUNK, ROW_CHUNK),
                sel.reshape(NW, N_GRP, 16))

    row_a, sel_a = tables(a_idx)
    row_b, sel_b = tables(b_idx)
    return row_a, sel_a, row_b, sel_b


_ROW_A, _SEL_A, _ROW_B, _SEL_B = _build_tables()


def _sc_gather_body(a_hbm, b_hbm, row_a, sel_a, row_b, sel_b,
                    out_a, out_b, idx_v, rows_v, sel_v, out_v, prep_v, sem):
    wid = lax.axis_index("s") * 2 + lax.axis_index("c")

    pltpu.sync_copy(row_a.at[wid], idx_v.at[0])
    pltpu.sync_copy(row_b.at[wid], idx_v.at[1])
    pltpu.sync_copy(sel_a.at[wid], sel_v.at[0])
    pltpu.sync_copy(sel_b.at[wid], sel_v.at[1])
    copies = []
    for side, src_hbm in enumerate((a_hbm, b_hbm)):
        for j in range(N_RCHUNK):
            copies.append(pltpu.async_copy(
                src_hbm.at[idx_v.at[side, j]],
                rows_v.at[pl.ds((side * N_RCHUNK + j) * ROW_CHUNK, ROW_CHUNK)],
                sem,
            ))
    for cp in copies:
        cp.wait()

    for side in range(2):
        for g in range(N_GRP):
            s = sel_v[side, g]
            r = lax.shift_right_logical(s, 4) + (side * PER_W)
            col = lax.bitwise_and(s, 15)
            out_v[pl.ds(side * PER_W + g * 16, 16)] = (
                plsc.load_gather(rows_v, [r, col]))

    for c in range(N_CH):
        pltpu.sync_copy(out_v.at[pl.ds(c * S_PER_W, S_PER_W)],
                        out_a.at[pl.ds(c * N_S + wid * S_PER_W, S_PER_W)])

    for k in range(S_PER_W // 16):
        v0 = out_v[pl.ds(PER_W + 0 * S_PER_W + k * 16, 16)]
        v1 = out_v[pl.ds(PER_W + 1 * S_PER_W + k * 16, 16)]
        v2 = out_v[pl.ds(PER_W + 2 * S_PER_W + k * 16, 16)]
        prep_v[pl.ds(0 * S_PER_W + k * 16, 16)] = -2.0 * v0
        prep_v[pl.ds(1 * S_PER_W + k * 16, 16)] = -2.0 * v1
        prep_v[pl.ds(2 * S_PER_W + k * 16, 16)] = -2.0 * v2
        prep_v[pl.ds(3 * S_PER_W + k * 16, 16)] = v0 * v0 + v1 * v1 + v2 * v2
    for c in range(4):
        pltpu.sync_copy(prep_v.at[pl.ds(c * S_PER_W, S_PER_W)],
                        out_b.at[pl.ds(c * N_S + wid * S_PER_W, S_PER_W)])


@functools.cache
def _sc_gather():
    return pl.kernel(
        _sc_gather_body,
        mesh=plsc.VectorSubcoreMesh(core_axis_name="c", subcore_axis_name="s"),
        compiler_params=pltpu.CompilerParams(
            needs_layout_passes=False, use_tc_tiling_on_sc=False),
        out_type=[
            jax.ShapeDtypeStruct((N_CH * N_S,), jnp.float32),
            jax.ShapeDtypeStruct((4 * N_S,), jnp.float32),
        ],
        scratch_types=[
            pltpu.VMEM((2, N_RCHUNK, ROW_CHUNK), jnp.int32),
            pltpu.VMEM((2 * PER_W, 16), jnp.float32),
            pltpu.VMEM((2, N_GRP, 16), jnp.int32),
            pltpu.VMEM((2 * PER_W,), jnp.float32),
            pltpu.VMEM((4 * S_PER_W,), jnp.float32),
            pltpu.SemaphoreType.DMA,
        ],
    )


A_BLK = 32
N_ABLK = N_S // A_BLK
N_BBLK = N_S // 128


def _tc_chamfer_body(a_ref, b_ref, out_ref, bb_ref):
    for c in range(4):
        bb_ref[c] = jnp.broadcast_to(b_ref[c:c + 1, :], (8, N_S))

    def blk_step(blk, vsum):
        base = blk * A_BLK
        grp = []
        accs = []
        for g in range(4):
            a0 = jnp.broadcast_to(
                a_ref[0, pl.ds(base + g * 8, 8), :], (8, 128))
            a1 = jnp.broadcast_to(
                a_ref[1, pl.ds(base + g * 8, 8), :], (8, 128))
            a2 = jnp.broadcast_to(
                a_ref[2, pl.ds(base + g * 8, 8), :], (8, 128))
            na = a0 * a0 + a1 * a1 + a2 * a2
            grp.append((a0, a1, a2, na))
            accs.append(jnp.full((8, 128), jnp.inf, dtype=jnp.float32))
        for bb in range(N_BBLK):
            b0 = bb_ref[0, :, pl.ds(bb * 128, 128)]
            b1 = bb_ref[1, :, pl.ds(bb * 128, 128)]
            b2 = bb_ref[2, :, pl.ds(bb * 128, 128)]
            nb = bb_ref[3, :, pl.ds(bb * 128, 128)]
            for g in range(4):
                a0, a1, a2, _ = grp[g]
                v = nb + b0 * a0 + b1 * a1 + b2 * a2
                accs[g] = jnp.minimum(accs[g], v)
        for g in range(4):
            r = accs[g] + grp[g][3]
            vsum = vsum + jnp.min(r, axis=1, keepdims=True)
        return vsum

    vsum = lax.fori_loop(0, N_ABLK, blk_step,
                         jnp.zeros((8, 1), dtype=jnp.float32))
    out_ref[0, 0] = jnp.sum(vsum)


_tc_chamfer = pl.pallas_call(
    _tc_chamfer_body,
    out_shape=jax.ShapeDtypeStruct((1, 1), jnp.float32),
    in_specs=[
        pl.BlockSpec(memory_space=pltpu.VMEM),
        pl.BlockSpec(memory_space=pltpu.VMEM),
    ],
    out_specs=pl.BlockSpec(memory_space=pltpu.SMEM),
    scratch_shapes=[pltpu.VMEM((4, 8, N_S), jnp.float32)],
)


@jax.jit
def kernel(a, b):
    a_flat = a.reshape(N_COLS * 6 // 16, 16)
    b_flat = b.reshape(N_COLS * 6 // 16, 16)
    ag_flat, bg_flat = _sc_gather()(
        a_flat, b_flat,
        jnp.asarray(_ROW_A), jnp.asarray(_SEL_A),
        jnp.asarray(_ROW_B), jnp.asarray(_SEL_B),
    )
    a_g = ag_flat.reshape(N_CH, N_S, 1)
    b_g = bg_flat.reshape(4, N_S)
    return _tc_chamfer(a_g, b_g)[0, 0]


# speedup vs baseline: 1.9600x; 1.2358x over previous
import functools

import numpy as np
import jax
import jax.numpy as jnp
from jax import lax
from jax.experimental import pallas as pl
from jax.experimental.pallas import tpu as pltpu
from jax.experimental.pallas import tpu_sc as plsc

N_COLS = 100000
N_S = 4096
N_CH = 3
NW = 32
S_PER_W = N_S // NW


def _build_tables():
    rng = np.random.default_rng(0)
    a_idx = rng.permutation(N_COLS)[:N_S].astype(np.int64)
    b_idx = rng.permutation(N_COLS)[:N_S].astype(np.int64)

    q = np.arange(S_PER_W, dtype=np.int64)
    s, rloc = q // 16, q % 16
    p_a = rloc * 8 + s
    gidx_a = np.empty((NW * N_CH, S_PER_W), np.int32)
    gidx_b = np.empty((NW * N_CH, S_PER_W), np.int32)
    for w in range(NW):
        for c in range(N_CH):
            gidx_a[w * N_CH + c] = c * N_COLS + a_idx[w * S_PER_W + p_a]
            gidx_b[w * N_CH + c] = c * N_COLS + b_idx[w * S_PER_W + q]
    return gidx_a, gidx_b


_GIDX_A, _GIDX_B = _build_tables()


def _sc_gather_body(a_hbm, b_hbm, gidx_a, gidx_b, out_a, out_b,
                    idx_v, vals_v, prep_v, gsem, osem):
    wid = lax.axis_index("s") * 2 + lax.axis_index("c")

    pltpu.sync_copy(gidx_a.at[pl.ds(wid * N_CH, N_CH)], idx_v.at[0])
    pltpu.sync_copy(gidx_b.at[pl.ds(wid * N_CH, N_CH)], idx_v.at[1])
    gathers = []
    for side, src in enumerate((a_hbm, b_hbm)):
        for c in range(N_CH):
            gathers.append(pltpu.async_copy(
                src.at[idx_v.at[side, c]], vals_v.at[side, c], gsem))
    for g in gathers:
        g.wait()

    outs = []
    for c in range(N_CH):
        for s in range(8):
            outs.append(pltpu.async_copy(
                vals_v.at[0, c, pl.ds(s * 16, 16)],
                out_a.at[c, s, pl.ds(wid * 16, 16)],
                osem))

    for k in range(S_PER_W // 16):
        v0 = vals_v[1, 0, pl.ds(k * 16, 16)]
        v1 = vals_v[1, 1, pl.ds(k * 16, 16)]
        v2 = vals_v[1, 2, pl.ds(k * 16, 16)]
        prep_v[0, pl.ds(k * 16, 16)] = -2.0 * v0
        prep_v[1, pl.ds(k * 16, 16)] = -2.0 * v1
        prep_v[2, pl.ds(k * 16, 16)] = -2.0 * v2
        prep_v[3, pl.ds(k * 16, 16)] = v0 * v0 + v1 * v1 + v2 * v2
    for c in range(4):
        outs.append(pltpu.async_copy(
            prep_v.at[c], out_b.at[c, pl.ds(wid * S_PER_W, S_PER_W)], osem))
    for o in outs:
        o.wait()


@functools.cache
def _sc_gather():
    return pl.kernel(
        _sc_gather_body,
        mesh=plsc.VectorSubcoreMesh(core_axis_name="c", subcore_axis_name="s"),
        compiler_params=pltpu.CompilerParams(
            needs_layout_passes=False, use_tc_tiling_on_sc=False),
        out_type=[
            jax.ShapeDtypeStruct((N_CH, 8, N_S // 8), jnp.float32),
            jax.ShapeDtypeStruct((4, N_S), jnp.float32),
        ],
        scratch_types=[
            pltpu.VMEM((2, N_CH, S_PER_W), jnp.int32),
            pltpu.VMEM((2, N_CH, S_PER_W), jnp.float32),
            pltpu.VMEM((4, S_PER_W), jnp.float32),
            pltpu.SemaphoreType.DMA,
            pltpu.SemaphoreType.DMA,
        ],
    )


A_BLK = 32
N_ABLK = N_S // A_BLK
N_BBLK = N_S // 128


def _tc_chamfer_body(a_ref, b_ref, out_ref, bb_ref, ab_ref, mb_ref):
    for c in range(4):
        bb_ref[c] = jnp.broadcast_to(b_ref[c:c + 1, :], (8, N_S))

    def abuild(k, carry):
        base = pl.multiple_of((k // 8) * 128, 128)
        sh = (k % 8) * 16
        for c in range(N_CH):
            av = pltpu.roll(a_ref[c, :, pl.ds(base, 128)], -sh, 1)
            for u in range(16):
                ab_ref[c, k * 16 + u] = jnp.broadcast_to(
                    av[:, u:u + 1], (8, 128))
        return carry

    lax.fori_loop(0, N_S // 8 // 16, abuild, 0)

    def blk_step(blk, carry):
        grp = []
        accs = []
        for g in range(4):
            rr = blk * 4 + g
            grp.append((ab_ref[0, rr], ab_ref[1, rr], ab_ref[2, rr]))
            accs.append(jnp.full((8, 128), jnp.inf, dtype=jnp.float32))
        def bb_step(bb, accs):
            off = pl.multiple_of(bb * 128, 128)
            b0 = bb_ref[0, :, pl.ds(off, 128)]
            b1 = bb_ref[1, :, pl.ds(off, 128)]
            b2 = bb_ref[2, :, pl.ds(off, 128)]
            nb = bb_ref[3, :, pl.ds(off, 128)]
            out = []
            for g in range(4):
                a0, a1, a2 = grp[g]
                v = nb + b0 * a0 + b1 * a1 + b2 * a2
                out.append(jnp.minimum(accs[g], v))
            return tuple(out)

        accs = lax.fori_loop(0, N_BBLK, bb_step, tuple(accs), unroll=4)
        for g in range(4):
            a0, a1, a2 = grp[g]
            na = a0 * a0 + a1 * a1 + a2 * a2
            mb_ref[blk * 4 + g] = accs[g] + na
        return carry

    lax.fori_loop(0, N_ABLK, blk_step, 0)

    def fin_step(i, sacc):
        return sacc + jnp.min(mb_ref[i], axis=1, keepdims=True)

    sacc = lax.fori_loop(0, N_S // 8, fin_step,
                         jnp.zeros((8, 1), dtype=jnp.float32), unroll=8)
    out_ref[0, 0] = jnp.sum(sacc)


_tc_chamfer = pl.pallas_call(
    _tc_chamfer_body,
    out_shape=jax.ShapeDtypeStruct((1, 1), jnp.float32),
    in_specs=[
        pl.BlockSpec(memory_space=pltpu.VMEM),
        pl.BlockSpec(memory_space=pltpu.VMEM),
    ],
    out_specs=pl.BlockSpec(memory_space=pltpu.SMEM),
    scratch_shapes=[
        pltpu.VMEM((4, 8, N_S), jnp.float32),
        pltpu.VMEM((N_CH, N_S // 8, 8, 128), jnp.float32),
        pltpu.VMEM((N_S // 8, 8, 128), jnp.float32),
    ],
)


@jax.jit
def kernel(a, b):
    a_flat = a.reshape(6 * N_COLS)
    b_flat = b.reshape(6 * N_COLS)
    a_g, b_g = _sc_gather()(
        a_flat, b_flat, jnp.asarray(_GIDX_A), jnp.asarray(_GIDX_B))
    return _tc_chamfer(a_g, b_g)[0, 0]


# speedup vs baseline: 2.0793x; 1.0609x over previous
import functools

import numpy as np
import jax
import jax.numpy as jnp
from jax import lax
from jax.experimental import pallas as pl
from jax.experimental.pallas import tpu as pltpu
from jax.experimental.pallas import tpu_sc as plsc

N_COLS = 100000
N_S = 4096
N_CH = 3
NW = 32
S_PER_W = N_S // NW


def _build_tables():
    rng = np.random.default_rng(0)
    a_idx = rng.permutation(N_COLS)[:N_S].astype(np.int64)
    b_idx = rng.permutation(N_COLS)[:N_S].astype(np.int64)

    q = np.arange(S_PER_W, dtype=np.int64)
    s, rloc = q // 16, q % 16
    p_a = rloc * 8 + s
    gidx_a = np.empty((NW * N_CH, S_PER_W), np.int32)
    gidx_b = np.empty((NW * N_CH, S_PER_W), np.int32)
    for w in range(NW):
        for c in range(N_CH):
            gidx_a[w * N_CH + c] = c * N_COLS + a_idx[w * S_PER_W + p_a]
            gidx_b[w * N_CH + c] = c * N_COLS + b_idx[w * S_PER_W + q]
    return gidx_a, gidx_b


_GIDX_A, _GIDX_B = _build_tables()


def _sc_gather_body(a_hbm, b_hbm, gidx_a, gidx_b, out_a, out_b,
                    idx_v, vals_v, prep_v, gsem, osem):
    wid = lax.axis_index("s") * 2 + lax.axis_index("c")

    pltpu.sync_copy(gidx_a.at[pl.ds(wid * N_CH, N_CH)], idx_v.at[0])
    pltpu.sync_copy(gidx_b.at[pl.ds(wid * N_CH, N_CH)], idx_v.at[1])
    gathers = []
    for side, src in enumerate((a_hbm, b_hbm)):
        for c in range(N_CH):
            gathers.append(pltpu.async_copy(
                src.at[idx_v.at[side, c]], vals_v.at[side, c], gsem))
    for g in gathers:
        g.wait()

    outs = []
    for c in range(N_CH):
        for s in range(8):
            outs.append(pltpu.async_copy(
                vals_v.at[0, c, pl.ds(s * 16, 16)],
                out_a.at[c, s, pl.ds(wid * 16, 16)],
                osem))

    for k in range(S_PER_W // 16):
        v0 = vals_v[1, 0, pl.ds(k * 16, 16)]
        v1 = vals_v[1, 1, pl.ds(k * 16, 16)]
        v2 = vals_v[1, 2, pl.ds(k * 16, 16)]
        prep_v[0, pl.ds(k * 16, 16)] = -2.0 * v0
        prep_v[1, pl.ds(k * 16, 16)] = -2.0 * v1
        prep_v[2, pl.ds(k * 16, 16)] = -2.0 * v2
        prep_v[3, pl.ds(k * 16, 16)] = v0 * v0 + v1 * v1 + v2 * v2
    for c in range(4):
        outs.append(pltpu.async_copy(
            prep_v.at[c], out_b.at[c, pl.ds(wid * S_PER_W, S_PER_W)], osem))
    for o in outs:
        o.wait()


@functools.cache
def _sc_gather():
    return pl.kernel(
        _sc_gather_body,
        mesh=plsc.VectorSubcoreMesh(core_axis_name="c", subcore_axis_name="s"),
        compiler_params=pltpu.CompilerParams(
            needs_layout_passes=False, use_tc_tiling_on_sc=False),
        out_type=[
            jax.ShapeDtypeStruct((N_CH, 8, N_S // 8), jnp.float32),
            jax.ShapeDtypeStruct((4, N_S), jnp.float32),
        ],
        scratch_types=[
            pltpu.VMEM((2, N_CH, S_PER_W), jnp.int32),
            pltpu.VMEM((2, N_CH, S_PER_W), jnp.float32),
            pltpu.VMEM((4, S_PER_W), jnp.float32),
            pltpu.SemaphoreType.DMA,
            pltpu.SemaphoreType.DMA,
        ],
    )


A_BLK = 32
N_ABLK = N_S // A_BLK
N_BBLK = N_S // 128


def _tc_chamfer_body(a_ref, b_ref, out_ref, bb_ref, ab_ref, mb_ref):
    for c in range(4):
        bb_ref[c] = jnp.broadcast_to(b_ref[c:c + 1, :], (8, N_S))

    def abuild(k2, carry):
        for half in range(2):
            k = k2 * 2 + half
            base = pl.multiple_of((k // 8) * 128, 128)
            sh = (k % 8) * 16
            for c in range(N_CH):
                av = pltpu.roll(a_ref[c, :, pl.ds(base, 128)], -sh, 1)
                for u in range(16):
                    ab_ref[c, k * 16 + u] = jnp.broadcast_to(
                        av[:, u:u + 1], (8, 128))
        return carry

    lax.fori_loop(0, N_S // 8 // 32, abuild, 0)

    def blk_step(blk, carry):
        grp = []
        accs = []
        for g in range(4):
            rr = blk * 4 + g
            grp.append((ab_ref[0, rr], ab_ref[1, rr], ab_ref[2, rr]))
            accs.append(jnp.full((8, 128), jnp.inf, dtype=jnp.float32))
        def bb_step(bb, accs):
            off = pl.multiple_of(bb * 128, 128)
            b0 = bb_ref[0, :, pl.ds(off, 128)]
            b1 = bb_ref[1, :, pl.ds(off, 128)]
            b2 = bb_ref[2, :, pl.ds(off, 128)]
            nb = bb_ref[3, :, pl.ds(off, 128)]
            out = []
            for g in range(4):
                a0, a1, a2 = grp[g]
                v = nb + b0 * a0 + b1 * a1 + b2 * a2
                out.append(jnp.minimum(accs[g], v))
            return tuple(out)

        accs = lax.fori_loop(0, N_BBLK, bb_step, tuple(accs), unroll=4)
        for g in range(4):
            a0, a1, a2 = grp[g]
            na = a0 * a0 + a1 * a1 + a2 * a2
            mb_ref[blk * 4 + g] = accs[g] + na
        return carry

    lax.fori_loop(0, N_ABLK, blk_step, 0)

    def fin_step(i, sacc):
        return sacc + jnp.min(mb_ref[i], axis=1, keepdims=True)

    sacc = lax.fori_loop(0, N_S // 8, fin_step,
                         jnp.zeros((8, 1), dtype=jnp.float32), unroll=16)
    out_ref[0, 0] = jnp.sum(sacc)


_tc_chamfer = pl.pallas_call(
    _tc_chamfer_body,
    out_shape=jax.ShapeDtypeStruct((1, 1), jnp.float32),
    in_specs=[
        pl.BlockSpec(memory_space=pltpu.VMEM),
        pl.BlockSpec(memory_space=pltpu.VMEM),
    ],
    out_specs=pl.BlockSpec(memory_space=pltpu.SMEM),
    scratch_shapes=[
        pltpu.VMEM((4, 8, N_S), jnp.float32),
        pltpu.VMEM((N_CH, N_S // 8, 8, 128), jnp.float32),
        pltpu.VMEM((N_S // 8, 8, 128), jnp.float32),
    ],
)


@jax.jit
def kernel(a, b):
    a_flat = a.reshape(6 * N_COLS)
    b_flat = b.reshape(6 * N_COLS)
    a_g, b_g = _sc_gather()(
        a_flat, b_flat, jnp.asarray(_GIDX_A), jnp.asarray(_GIDX_B))
    return _tc_chamfer(a_g, b_g)[0, 0]
